# baseline (speedup 1.0000x reference)
import functools

import jax
import jax.numpy as jnp
from jax import lax
from jax.experimental import pallas as pl
from jax.experimental.pallas import tpu as pltpu
from jax.experimental.pallas import tpu_sc as plsc

_K = 8192
_D = 256
_BETA = 0.25

_M_BLK = 1024
_K_BLK = 4096
_BIG = 1e9


def _argmin_body(flat_ref, cb_ref, csqh_ref, iota_ref, idx_ref, loss_ref,
                 maxv_ref, mini_ref):
    j = pl.program_id(0)
    i = pl.program_id(1)
    nj = pl.num_programs(0)
    flat = flat_ref[...]
    cb = cb_ref[...]
    zsqh = jnp.sum(flat * flat, axis=1, keepdims=True) * -0.5
    r = lax.dot_general(flat, cb, (((1,), (1,)), ((), ())),
                        preferred_element_type=jnp.float32)
    dh = (zsqh + r) + csqh_ref[...]
    bmax = jnp.max(dh, axis=1, keepdims=True)
    cand = jnp.min(jnp.where(dh == bmax, iota_ref[...], _BIG),
                   axis=1, keepdims=True)

    @pl.when(j == 0)
    def _():
        maxv_ref[i] = bmax
        mini_ref[i] = cand

    @pl.when(j > 0)
    def _():
        better = bmax > maxv_ref[i]
        mini_ref[i] = jnp.where(better, cand, mini_ref[i])
        maxv_ref[i] = jnp.where(better, bmax, maxv_ref[i])

    @pl.when(j == nj - 1)
    def _():
        idx_ref[...] = mini_ref[i].astype(jnp.int32)
        loss_ref[...] = (jnp.sum(maxv_ref[i]) * -2.0).reshape(1, 1, 1)


def _tc_argmin(flat, codebook, csqh, iota_f):
    n = flat.shape[0]
    ni = n // _M_BLK
    nj = _K // _K_BLK
    idx, loss = pl.pallas_call(
        _argmin_body,
        grid=(nj, ni),
        in_specs=[
            pl.BlockSpec((_M_BLK, _D), lambda j, i: (i, 0)),
            pl.BlockSpec((_K_BLK, _D), lambda j, i: (j, 0)),
            pl.BlockSpec((1, _K_BLK), lambda j, i: (0, j)),
            pl.BlockSpec((1, _K_BLK), lambda j, i: (0, j)),
        ],
        out_specs=[
            pl.BlockSpec((_M_BLK, 1), lambda j, i: (i, 0)),
            pl.BlockSpec((1, 1, 1), lambda j, i: (i, 0, 0)),
        ],
        out_shape=[
            jax.ShapeDtypeStruct((n, 1), jnp.int32),
            jax.ShapeDtypeStruct((ni, 1, 1), jnp.float32),
        ],
        scratch_shapes=[
            pltpu.VMEM((ni, _M_BLK, 1), jnp.float32),
            pltpu.VMEM((ni, _M_BLK, 1), jnp.float32),
        ],
    )(flat, codebook, csqh, iota_f)
    return idx[:, 0], jnp.sum(loss)


def _sc_gather(codebook, idx):
    info = plsc.get_sparse_core_info()
    nc, ns = info.num_cores, info.num_subcores
    nw = nc * ns
    b = idx.shape[0]
    b_per_w = b // nw
    ch = 128
    mesh = plsc.VectorSubcoreMesh(core_axis_name="c", subcore_axis_name="s")

    nch = b_per_w // ch

    @functools.partial(
        pl.kernel, mesh=mesh,
        out_type=jax.ShapeDtypeStruct((b, _D), jnp.float32),
        scratch_types=[
            pltpu.VMEM((b_per_w,), jnp.int32),
            pltpu.VMEM((ch, _D), jnp.float32),
            pltpu.VMEM((ch, _D), jnp.float32),
            pltpu.SemaphoreType.DMA,
            pltpu.SemaphoreType.DMA,
        ],
    )
    def gk(table_hbm, idx_hbm, out_hbm, idx_v, rows0, rows1, sem0, sem1):
        wid = lax.axis_index("s") * nc + lax.axis_index("c")
        base = wid * b_per_w
        bufs = (rows0, rows1)
        sems = (sem0, sem1)
        pltpu.sync_copy(idx_hbm.at[pl.ds(base, b_per_w)], idx_v)
        copies = []
        for c in range(nch):
            copies.append(pltpu.async_copy(
                table_hbm.at[idx_v.at[pl.ds(c * ch, ch)]],
                bufs[c % 2], sems[c % 2]))
            if c >= 1:
                copies[c - 1].wait()
                pltpu.sync_copy(bufs[(c - 1) % 2],
                                out_hbm.at[pl.ds(base + (c - 1) * ch, ch)])
        copies[nch - 1].wait()
        pltpu.sync_copy(bufs[(nch - 1) % 2],
                        out_hbm.at[pl.ds(base + (nch - 1) * ch, ch)])

    return gk(codebook, idx)


def kernel(z_e, codebook):
    flat = z_e.reshape(-1, _D)
    n = flat.shape[0]
    csqh = jnp.sum(codebook ** 2, axis=-1, keepdims=True).T * -0.5
    iota_f = jnp.arange(_K, dtype=jnp.float32).reshape(1, _K)
    idx, loss_sum = _tc_argmin(flat, codebook, csqh, iota_f)
    z_q = _sc_gather(codebook, idx).reshape(z_e.shape)
    vq_loss = (1.0 + _BETA) * loss_sum / (n * _D)
    indices = idx.reshape(z_e.shape[:-1])
    return indices, z_q, vq_loss


# speedup vs baseline: 1.0112x; 1.0112x over previous
import functools

import jax
import jax.numpy as jnp
from jax import lax
from jax.experimental import pallas as pl
from jax.experimental.pallas import tpu as pltpu
from jax.experimental.pallas import tpu_sc as plsc

_K = 8192
_D = 256
_BETA = 0.25

_M_BLK = 2048
_K_BLK = 2048
_BIG = 1e9


def _argmin_body(flat_ref, cb_ref, csqh_ref, iota_ref, idx_ref, loss_ref,
                 maxv_ref, mini_ref):
    j = pl.program_id(0)
    i = pl.program_id(1)
    nj = pl.num_programs(0)
    flat = flat_ref[...]
    cb = cb_ref[...]
    zsqh = jnp.sum(flat * flat, axis=1, keepdims=True) * -0.5
    r = lax.dot_general(flat, cb, (((1,), (1,)), ((), ())),
                        preferred_element_type=jnp.float32)
    dh = (zsqh + r) + csqh_ref[...]
    bmax = jnp.max(dh, axis=1, keepdims=True)
    cand = jnp.min(jnp.where(dh == bmax, iota_ref[...], _BIG),
                   axis=1, keepdims=True)

    @pl.when(j == 0)
    def _():
        maxv_ref[i] = bmax
        mini_ref[i] = cand

    @pl.when(j > 0)
    def _():
        better = bmax > maxv_ref[i]
        mini_ref[i] = jnp.where(better, cand, mini_ref[i])
        maxv_ref[i] = jnp.where(better, bmax, maxv_ref[i])

    @pl.when(j == nj - 1)
    def _():
        idx_ref[...] = mini_ref[i].astype(jnp.int32)
        loss_ref[...] = (jnp.sum(maxv_ref[i]) * -2.0).reshape(1, 1, 1)


def _tc_argmin(flat, codebook, csqh, iota_f):
    n = flat.shape[0]
    ni = n // _M_BLK
    nj = _K // _K_BLK
    idx, loss = pl.pallas_call(
        _argmin_body,
        grid=(nj, ni),
        in_specs=[
            pl.BlockSpec((_M_BLK, _D), lambda j, i: (i, 0)),
            pl.BlockSpec((_K_BLK, _D), lambda j, i: (j, 0)),
            pl.BlockSpec((1, _K_BLK), lambda j, i: (0, j)),
            pl.BlockSpec((1, _K_BLK), lambda j, i: (0, j)),
        ],
        out_specs=[
            pl.BlockSpec((_M_BLK, 1), lambda j, i: (i, 0)),
            pl.BlockSpec((1, 1, 1), lambda j, i: (i, 0, 0)),
        ],
        out_shape=[
            jax.ShapeDtypeStruct((n, 1), jnp.int32),
            jax.ShapeDtypeStruct((ni, 1, 1), jnp.float32),
        ],
        scratch_shapes=[
            pltpu.VMEM((ni, _M_BLK, 1), jnp.float32),
            pltpu.VMEM((ni, _M_BLK, 1), jnp.float32),
        ],
    )(flat, codebook, csqh, iota_f)
    return idx[:, 0], jnp.sum(loss)


def _sc_gather(codebook, idx):
    info = plsc.get_sparse_core_info()
    nc, ns = info.num_cores, info.num_subcores
    nw = nc * ns
    b = idx.shape[0]
    b_per_w = b // nw
    ch = 128
    mesh = plsc.VectorSubcoreMesh(core_axis_name="c", subcore_axis_name="s")

    nch = b_per_w // ch

    @functools.partial(
        pl.kernel, mesh=mesh,
        out_type=jax.ShapeDtypeStruct((b, _D), jnp.float32),
        scratch_types=[
            pltpu.VMEM((b_per_w,), jnp.int32),
            pltpu.VMEM((ch, _D), jnp.float32),
            pltpu.VMEM((ch, _D), jnp.float32),
            pltpu.SemaphoreType.DMA,
            pltpu.SemaphoreType.DMA,
        ],
    )
    def gk(table_hbm, idx_hbm, out_hbm, idx_v, rows0, rows1, sem0, sem1):
        wid = lax.axis_index("s") * nc + lax.axis_index("c")
        base = wid * b_per_w
        bufs = (rows0, rows1)
        sems = (sem0, sem1)
        pltpu.sync_copy(idx_hbm.at[pl.ds(base, b_per_w)], idx_v)
        copies = []
        for c in range(nch):
            copies.append(pltpu.async_copy(
                table_hbm.at[idx_v.at[pl.ds(c * ch, ch)]],
                bufs[c % 2], sems[c % 2]))
            if c >= 1:
                copies[c - 1].wait()
                pltpu.sync_copy(bufs[(c - 1) % 2],
                                out_hbm.at[pl.ds(base + (c - 1) * ch, ch)])
        copies[nch - 1].wait()
        pltpu.sync_copy(bufs[(nch - 1) % 2],
                        out_hbm.at[pl.ds(base + (nch - 1) * ch, ch)])

    return gk(codebook, idx)


def kernel(z_e, codebook):
    flat = z_e.reshape(-1, _D)
    n = flat.shape[0]
    csqh = jnp.sum(codebook ** 2, axis=-1, keepdims=True).T * -0.5
    iota_f = jnp.arange(_K, dtype=jnp.float32).reshape(1, _K)
    idx, loss_sum = _tc_argmin(flat, codebook, csqh, iota_f)
    z_q = _sc_gather(codebook, idx).reshape(z_e.shape)
    vq_loss = (1.0 + _BETA) * loss_sum / (n * _D)
    indices = idx.reshape(z_e.shape[:-1])
    return indices, z_q, vq_loss
